# Initial kernel scaffold; baseline (speedup 1.0000x reference)
import jax
import jax.numpy as jnp
from jax.experimental import pallas as pl


def kernel(x, edge_index, batch, W1, att_src1, att_dst1, b1, W2, att_src2, att_dst2, b2):
    raise NotImplementedError("write your pallas kernel here")


# speedup vs baseline: 1.1364x; 1.1364x over previous
import functools

import jax
import jax.numpy as jnp
from jax import lax
from jax.experimental import pallas as pl
from jax.experimental.pallas import tpu as pltpu

N = 10000
E = 320000
IN = 128
HID = 128
HEADS = 8
OUT = 128
G = 64

ROWS = 1000


def _dense1_body(x_ref, w_ref, asrc_w_ref, adst_w_ref, h_ref, asrc_ref, adst_ref):
    h = jnp.dot(x_ref[...], w_ref[...], preferred_element_type=jnp.float32)
    h_ref[...] = h
    hh = h.reshape(ROWS, HEADS, HID)
    asrc_ref[...] = jnp.sum(hh * asrc_w_ref[...][None], axis=-1)
    adst_ref[...] = jnp.sum(hh * adst_w_ref[...][None], axis=-1)


def _dense1(x, W1, att_src1, att_dst1):
    grid = N // ROWS
    return pl.pallas_call(
        _dense1_body,
        grid=(grid,),
        in_specs=[
            pl.BlockSpec((ROWS, IN), lambda i: (i, 0)),
            pl.BlockSpec((IN, HEADS * HID), lambda i: (0, 0)),
            pl.BlockSpec((HEADS, HID), lambda i: (0, 0)),
            pl.BlockSpec((HEADS, HID), lambda i: (0, 0)),
        ],
        out_specs=[
            pl.BlockSpec((ROWS, HEADS * HID), lambda i: (i, 0)),
            pl.BlockSpec((ROWS, HEADS), lambda i: (i, 0)),
            pl.BlockSpec((ROWS, HEADS), lambda i: (i, 0)),
        ],
        out_shape=[
            jax.ShapeDtypeStruct((N, HEADS * HID), jnp.float32),
            jax.ShapeDtypeStruct((N, HEADS), jnp.float32),
            jax.ShapeDtypeStruct((N, HEADS), jnp.float32),
        ],
    )(x, W1, att_src1, att_dst1)


def _dense2_body(acc_ref, den_ref, b1_ref, w2_ref, as2_ref, ad2_ref,
                 h2_ref, asrc2_ref, adst2_ref):
    acc = acc_ref[...]
    den = den_ref[...]
    hn = acc / (den + 1e-16).reshape(ROWS, HEADS, 1)
    h1 = hn.reshape(ROWS, HEADS * HID) + b1_ref[...][None]
    h1 = jnp.where(h1 > 0, h1, jnp.exp(jnp.minimum(h1, 0.0)) - 1.0)
    h2 = jnp.dot(h1, w2_ref[...], preferred_element_type=jnp.float32)
    h2_ref[...] = h2
    asrc2_ref[...] = jnp.sum(h2 * as2_ref[...], axis=-1, keepdims=True)
    adst2_ref[...] = jnp.sum(h2 * ad2_ref[...], axis=-1, keepdims=True)


def _dense2(acc1, den1, b1, W2, att_src2, att_dst2):
    grid = N // ROWS
    return pl.pallas_call(
        _dense2_body,
        grid=(grid,),
        in_specs=[
            pl.BlockSpec((ROWS, HEADS, HID), lambda i: (i, 0, 0)),
            pl.BlockSpec((ROWS, HEADS), lambda i: (i, 0)),
            pl.BlockSpec((HEADS * HID,), lambda i: (0,)),
            pl.BlockSpec((HEADS * HID, OUT), lambda i: (0, 0)),
            pl.BlockSpec((1, OUT), lambda i: (0, 0)),
            pl.BlockSpec((1, OUT), lambda i: (0, 0)),
        ],
        out_specs=[
            pl.BlockSpec((ROWS, OUT), lambda i: (i, 0)),
            pl.BlockSpec((ROWS, 1), lambda i: (i, 0)),
            pl.BlockSpec((ROWS, 1), lambda i: (i, 0)),
        ],
        out_shape=[
            jax.ShapeDtypeStruct((N, OUT), jnp.float32),
            jax.ShapeDtypeStruct((N, 1), jnp.float32),
            jax.ShapeDtypeStruct((N, 1), jnp.float32),
        ],
    )(acc1, den1, b1, W2, att_src2, att_dst2)


def _pool_body(acc_ref, den_ref, b2_ref, batch_ref, out_ref):
    i = pl.program_id(0)
    h2 = acc_ref[...] / (den_ref[...] + 1e-16) + b2_ref[...][None]
    seg = batch_ref[...].reshape(1, ROWS)
    gid = lax.broadcasted_iota(jnp.int32, (G, ROWS), 0)
    onehot = (seg == gid).astype(jnp.float32)
    sums = jnp.dot(onehot, h2, preferred_element_type=jnp.float32)
    counts = jnp.sum(onehot, axis=1, keepdims=True)

    @pl.when(i == 0)
    def _init():
        out_ref[...] = jnp.zeros_like(out_ref)

    out_ref[:, :OUT] += sums
    out_ref[:, OUT:] += counts


def _pool(acc2, den2, b2, batch):
    grid = N // ROWS
    out = pl.pallas_call(
        _pool_body,
        grid=(grid,),
        in_specs=[
            pl.BlockSpec((ROWS, OUT), lambda i: (i, 0)),
            pl.BlockSpec((ROWS, 1), lambda i: (i, 0)),
            pl.BlockSpec((OUT,), lambda i: (0,)),
            pl.BlockSpec((1, 1, ROWS), lambda i: (i, 0, 0)),
        ],
        out_specs=pl.BlockSpec((G, OUT + 1), lambda i: (0, 0)),
        out_shape=jax.ShapeDtypeStruct((G, OUT + 1), jnp.float32),
    )(acc2, den2, b2, batch.reshape(N // ROWS, 1, ROWS))
    sums = out[:, :OUT]
    counts = out[:, OUT:]
    return sums / jnp.clip(counts, 1.0)


NW = 32
NPW = 320
N_PAD = NW * NPW
RW = 144
CAP = 12800
PC = 128
SC_CHUNK = 2000


def _sc_edge_kernel(heads):
    from jax.experimental.pallas import tpu_sc as plsc

    mesh = plsc.VectorSubcoreMesh(core_axis_name="c", subcore_axis_name="s")

    def body(src_hbm, dst_hbm, htab_hbm, asrc_hbm, adst_hbm, zeros_hbm, acc_hbm,
             sbuf_src, sbuf_dst, src_c, dstl_c, asrc_t, adst_t, out_l, hbuf,
             sem0, sem1):
        wid = lax.axis_index("s") * 2 + lax.axis_index("c")
        lo = wid * NPW

        zvi = jnp.zeros((16,), jnp.int32)

        def zloop(i, _):
            src_c[pl.ds(pl.multiple_of(i * 16, 16), 16)] = zvi
            dstl_c[pl.ds(pl.multiple_of(i * 16, 16), 16)] = zvi
            return 0

        lax.fori_loop(0, CAP // 16, zloop, 0)

        def chunk_scan(ci, pos):
            base = ci * SC_CHUNK
            pltpu.sync_copy(src_hbm.at[pl.ds(base, SC_CHUNK)], sbuf_src)
            pltpu.sync_copy(dst_hbm.at[pl.ds(base, SC_CHUNK)], sbuf_dst)

            def group(gi, pos):
                off = pl.multiple_of(gi * 16, 16)
                s16 = sbuf_src[pl.ds(off, 16)]
                d16 = sbuf_dst[pl.ds(off, 16)]
                m = (d16 >= lo) & (d16 < lo + NPW)
                p = jnp.minimum(pos, CAP - 16)
                plsc.store_compressed(src_c.at[pl.ds(p, 16)], s16, m)
                plsc.store_compressed(dstl_c.at[pl.ds(p, 16)], d16 - lo, m)
                cnt = plsc.all_reduce_population_count(m)
                return pos + cnt[0]

            return lax.fori_loop(0, SC_CHUNK // 16, group, pos)

        count = lax.fori_loop(0, E // SC_CHUNK, chunk_scan, jnp.int32(0))
        nchunks = (count + PC - 1) // PC

        def gather_desc(k, ch, b, sem):
            idx = src_c.at[pl.ds(ch * PC, PC)]
            return pltpu.make_async_copy(htab_hbm.at[k].at[idx], hbuf.at[b], sem)

        def head_body(k, _):
            pltpu.sync_copy(asrc_hbm.at[k], asrc_t)
            pltpu.sync_copy(adst_hbm.at[k, pl.ds(lo, NPW)], adst_t)
            pltpu.sync_copy(zeros_hbm, out_l)

            @pl.when(nchunks > 0)
            def _p0():
                gather_desc(k, 0, 0, sem0).start()

            @pl.when(nchunks > 1)
            def _p1():
                gather_desc(k, 1, 1, sem1).start()

            def process(ch, b, sem):
                gather_desc(k, ch, b, sem).wait()

                @pl.when(ch + 2 < nchunks)
                def _nxt():
                    gather_desc(k, ch + 2, b, sem).start()

                def group(g, _):
                    off = pl.multiple_of(ch * PC + g * 16, 16)
                    s16 = src_c[pl.ds(off, 16)]
                    d16 = dstl_c[pl.ds(off, 16)]
                    a = plsc.load_gather(asrc_t, [s16])
                    bb = plsc.load_gather(adst_t, [d16])
                    e = a + bb
                    e = jnp.where(e >= 0, e, 0.2 * e)
                    w16 = jnp.exp(e)
                    eidx = off + lax.broadcasted_iota(jnp.int32, (16,), 0)
                    w16 = jnp.where(eidx < count, w16, 0.0)
                    for i in range(16):
                        wi = w16[i]
                        di = d16[i]
                        erow = g * 16 + i
                        for j in range(RW // 16):
                            hv = hbuf[b, erow, pl.ds(j * 16, 16)]
                            plsc.addupdate(out_l.at[di, pl.ds(j * 16, 16)], hv * wi)
                    return 0

                lax.fori_loop(0, PC // 16, group, 0)

            def outer(it, _):
                ch0 = it * 2

                @pl.when(ch0 < nchunks)
                def _b0():
                    process(ch0, 0, sem0)

                @pl.when(ch0 + 1 < nchunks)
                def _b1():
                    process(ch0 + 1, 1, sem1)

                return 0

            lax.fori_loop(0, (nchunks + 1) // 2, outer, 0)
            pltpu.sync_copy(out_l, acc_hbm.at[k, pl.ds(lo, NPW)])
            return 0

        lax.fori_loop(0, heads, head_body, 0)

    return functools.partial(
        pl.kernel, body,
        out_type=jax.ShapeDtypeStruct((heads, N_PAD, RW), jnp.float32),
        mesh=mesh,
        scratch_types=[
            pltpu.VMEM((SC_CHUNK,), jnp.int32),
            pltpu.VMEM((SC_CHUNK,), jnp.int32),
            pltpu.VMEM((CAP,), jnp.int32),
            pltpu.VMEM((CAP,), jnp.int32),
            pltpu.VMEM((N_PAD,), jnp.float32),
            pltpu.VMEM((NPW,), jnp.float32),
            pltpu.VMEM((NPW, RW), jnp.float32),
            pltpu.VMEM((2, PC, RW), jnp.float32),
            pltpu.SemaphoreType.DMA,
            pltpu.SemaphoreType.DMA,
        ],
    )()


def _edge_phase_sc(htab, asrc_t, adst_t, src, dst):
    heads = htab.shape[0]
    zeros = jnp.zeros((NPW, RW), jnp.float32)
    return _sc_edge_kernel(heads)(src, dst, htab, asrc_t, adst_t, zeros)


def _edge_phase(h, asrc, adst, src, dst, heads, ch):
    e = asrc[src] + adst[dst]
    e = jnp.where(e > 0, e, 0.2 * e)
    w = jnp.exp(e)
    den = jax.ops.segment_sum(w, dst, num_segments=N)
    msg = h.reshape(N, heads, ch)[src] * w[:, :, None]
    acc = jax.ops.segment_sum(msg, dst, num_segments=N)
    return acc, den


def kernel(x, edge_index, batch, W1, att_src1, att_dst1, b1, W2, att_src2, att_dst2, b2):
    src = edge_index[0]
    dst = edge_index[1]
    h1, asrc1, adst1 = _dense1(x, W1, att_src1, att_dst1)
    acc1, den1 = _edge_phase(h1, asrc1, adst1, src, dst, HEADS, HID)
    h2, asrc2, adst2 = _dense2(acc1, den1, b1, W2, att_src2, att_dst2)
    acc2, den2 = _edge_phase(h2, asrc2, adst2, src, dst, 1, OUT)
    return _pool(acc2.reshape(N, OUT), den2, b2, batch)


# speedup vs baseline: 5.4571x; 4.8020x over previous
import functools

import jax
import jax.numpy as jnp
from jax import lax
from jax.experimental import pallas as pl
from jax.experimental.pallas import tpu as pltpu

N = 10000
E = 320000
IN = 128
HID = 128
HEADS = 8
OUT = 128
G = 64

ROWS = 1000


def _dense1_body(x_ref, w_ref, asrc_w_ref, adst_w_ref, h_ref, asrc_ref, adst_ref):
    h = jnp.dot(x_ref[...], w_ref[...], preferred_element_type=jnp.float32)
    h_ref[...] = h
    hh = h.reshape(ROWS, HEADS, HID)
    asrc_ref[...] = jnp.sum(hh * asrc_w_ref[...][None], axis=-1)
    adst_ref[...] = jnp.sum(hh * adst_w_ref[...][None], axis=-1)


def _dense1(x, W1, att_src1, att_dst1):
    grid = N // ROWS
    return pl.pallas_call(
        _dense1_body,
        grid=(grid,),
        in_specs=[
            pl.BlockSpec((ROWS, IN), lambda i: (i, 0)),
            pl.BlockSpec((IN, HEADS * HID), lambda i: (0, 0)),
            pl.BlockSpec((HEADS, HID), lambda i: (0, 0)),
            pl.BlockSpec((HEADS, HID), lambda i: (0, 0)),
        ],
        out_specs=[
            pl.BlockSpec((ROWS, HEADS * HID), lambda i: (i, 0)),
            pl.BlockSpec((ROWS, HEADS), lambda i: (i, 0)),
            pl.BlockSpec((ROWS, HEADS), lambda i: (i, 0)),
        ],
        out_shape=[
            jax.ShapeDtypeStruct((N, HEADS * HID), jnp.float32),
            jax.ShapeDtypeStruct((N, HEADS), jnp.float32),
            jax.ShapeDtypeStruct((N, HEADS), jnp.float32),
        ],
    )(x, W1, att_src1, att_dst1)


def _dense2_body(acc_ref, den_ref, b1_ref, w2_ref, as2_ref, ad2_ref,
                 h2_ref, asrc2_ref, adst2_ref):
    a = acc_ref[...]
    hn = a / (den_ref[...][:, :, 0:1] + 1e-16)
    h1 = jnp.concatenate([hn[k] for k in range(HEADS)], axis=-1)
    h1 = h1 + b1_ref[...][None]
    h1 = jnp.where(h1 > 0, h1, jnp.exp(jnp.minimum(h1, 0.0)) - 1.0)
    h2 = jnp.dot(h1, w2_ref[...], preferred_element_type=jnp.float32)
    h2_ref[...] = h2
    asrc2_ref[...] = jnp.sum(h2 * as2_ref[...], axis=-1, keepdims=True)
    adst2_ref[...] = jnp.sum(h2 * ad2_ref[...], axis=-1, keepdims=True)


def _dense2(acc1, den1, b1, W2, att_src2, att_dst2):
    grid = N // ROWS
    return pl.pallas_call(
        _dense2_body,
        grid=(grid,),
        in_specs=[
            pl.BlockSpec((HEADS, ROWS, HID), lambda i: (0, i, 0)),
            pl.BlockSpec((HEADS, ROWS, 16), lambda i: (0, i, 0)),
            pl.BlockSpec((HEADS * HID,), lambda i: (0,)),
            pl.BlockSpec((HEADS * HID, OUT), lambda i: (0, 0)),
            pl.BlockSpec((1, OUT), lambda i: (0, 0)),
            pl.BlockSpec((1, OUT), lambda i: (0, 0)),
        ],
        out_specs=[
            pl.BlockSpec((ROWS, OUT), lambda i: (i, 0)),
            pl.BlockSpec((ROWS, 1), lambda i: (i, 0)),
            pl.BlockSpec((ROWS, 1), lambda i: (i, 0)),
        ],
        out_shape=[
            jax.ShapeDtypeStruct((N, OUT), jnp.float32),
            jax.ShapeDtypeStruct((N, 1), jnp.float32),
            jax.ShapeDtypeStruct((N, 1), jnp.float32),
        ],
    )(acc1, den1, b1, W2, att_src2, att_dst2)


def _pool_body(acc_ref, den_ref, b2_ref, batch_ref, out_ref):
    i = pl.program_id(0)
    a = acc_ref[...]
    h2 = a[0] / (den_ref[...][0, :, 0:1] + 1e-16) + b2_ref[...][None]
    seg = batch_ref[...].reshape(1, ROWS)
    gid = lax.broadcasted_iota(jnp.int32, (G, ROWS), 0)
    onehot = (seg == gid).astype(jnp.float32)
    sums = jnp.dot(onehot, h2, preferred_element_type=jnp.float32)
    counts = jnp.sum(onehot, axis=1, keepdims=True)

    @pl.when(i == 0)
    def _init():
        out_ref[...] = jnp.zeros_like(out_ref)

    out_ref[:, :OUT] += sums
    out_ref[:, OUT:] += counts


def _pool(acc2, den2, b2, batch):
    grid = N // ROWS
    out = pl.pallas_call(
        _pool_body,
        grid=(grid,),
        in_specs=[
            pl.BlockSpec((1, ROWS, OUT), lambda i: (0, i, 0)),
            pl.BlockSpec((1, ROWS, 16), lambda i: (0, i, 0)),
            pl.BlockSpec((OUT,), lambda i: (0,)),
            pl.BlockSpec((1, 1, ROWS), lambda i: (i, 0, 0)),
        ],
        out_specs=pl.BlockSpec((G, OUT + 1), lambda i: (0, 0)),
        out_shape=jax.ShapeDtypeStruct((G, OUT + 1), jnp.float32),
    )(acc2, den2, b2, batch.reshape(N // ROWS, 1, ROWS))
    sums = out[:, :OUT]
    counts = out[:, OUT:]
    return sums / jnp.clip(counts, 1.0)


NW = 32
NPW = 320
N_PAD = NW * NPW
RW = 128
CAPL = 960
CAP = 16 * CAPL
PC = 128
SC_CHUNK = 2000


def _sc_edge_kernel(heads):
    from jax.experimental.pallas import tpu_sc as plsc

    mesh = plsc.VectorSubcoreMesh(core_axis_name="c", subcore_axis_name="s")

    def body(src_hbm, dst_hbm, htab_hbm, asrc_hbm, adst_hbm, zeros_hbm,
             acc_hbm, den_hbm,
             sbuf_src, sbuf_dst, src_c, dstl_c, idxb, asrc_t, adst_t,
             out_l, den_l, hbuf, sem0, sem1):
        wid = lax.axis_index("s") * 2 + lax.axis_index("c")
        lo = wid * NPW

        zvi = jnp.zeros((16,), jnp.int32)
        sentinel = jnp.full((16,), NPW, jnp.int32)

        def zloop(i, _):
            src_c[pl.ds(pl.multiple_of(i * 16, 16), 16)] = zvi
            dstl_c[pl.ds(pl.multiple_of(i * 16, 16), 16)] = sentinel
            return 0

        lax.fori_loop(0, CAP // 16, zloop, 0)

        lane = lax.broadcasted_iota(jnp.int32, (16,), 0)
        ones = jnp.full((16,), 1, jnp.int32)
        zeros16 = jnp.zeros((16,), jnp.int32)

        def chunk_scan(ci, pos_v):
            base = pl.multiple_of(ci * SC_CHUNK, 8)
            pltpu.sync_copy(src_hbm.at[pl.ds(base, SC_CHUNK)], sbuf_src)
            pltpu.sync_copy(dst_hbm.at[pl.ds(base, SC_CHUNK)], sbuf_dst)

            def group(gi, pos_v):
                off = pl.multiple_of(gi * 16, 16)
                s16 = sbuf_src[pl.ds(off, 16)]
                d16 = sbuf_dst[pl.ds(off, 16)]
                m = (d16 >= lo) & (d16 < lo + NPW)
                addr = jnp.minimum(pos_v, CAPL - 1) * 16 + lane
                plsc.store_scatter(src_c, [addr], s16, mask=m)
                plsc.store_scatter(dstl_c, [addr], d16 - lo, mask=m)
                return pos_v + jnp.where(m, ones, zeros16)

            return lax.fori_loop(0, SC_CHUNK // 16, group, pos_v)

        pos_v = lax.fori_loop(0, E // SC_CHUNK, chunk_scan, zeros16)

        maxc = jnp.int32(0)
        for bit in (512, 256, 128, 64, 32, 16, 8, 4, 2, 1):
            t = maxc + bit
            c = plsc.all_reduce_population_count(pos_v >= t)
            maxc = jnp.where(c[0] > 0, t, maxc)
        maxc = jnp.minimum(maxc, CAPL)
        nchunks = (maxc + PC // 16 - 1) // (PC // 16)

        def issue_gather(k, ch, b, sem):
            cb = pl.multiple_of(ch * PC, 16)
            kbase = k * N_PAD
            for q in range(PC // 16):
                idxb[b, pl.ds(q * 16, 16)] = src_c[pl.ds(cb + q * 16, 16)] + kbase
            pltpu.make_async_copy(htab_hbm.at[idxb.at[b]], hbuf.at[b], sem).start()

        def wait_gather(k, ch, b, sem):
            pltpu.make_async_copy(htab_hbm.at[idxb.at[b]], hbuf.at[b], sem).wait()

        def head_body(k, _):
            kp = pl.multiple_of(k * N_PAD, 8)
            pltpu.sync_copy(asrc_hbm.at[pl.ds(kp, N_PAD)], asrc_t)
            pltpu.sync_copy(adst_hbm.at[pl.ds(kp + lo, NPW)], adst_t.at[pl.ds(0, NPW)])
            adst_t[pl.ds(NPW, 16)] = jnp.zeros((16,), jnp.float32)
            pltpu.sync_copy(zeros_hbm, out_l)
            pltpu.sync_copy(zeros_hbm.at[pl.ds(0, (NPW + 1) * 16)], den_l)

            @pl.when(nchunks > 0)
            def _p0():
                issue_gather(k, 0, 0, sem0)

            @pl.when(nchunks > 1)
            def _p1():
                issue_gather(k, 1, 1, sem1)

            def process(ch, b, sem):
                wait_gather(k, ch, b, sem)

                def group(g, _):
                    off = pl.multiple_of(ch * PC + g * 16, 16)
                    s16 = src_c[pl.ds(off, 16)]
                    d16 = dstl_c[pl.ds(off, 16)]
                    a = plsc.load_gather(asrc_t, [s16])
                    bb = plsc.load_gather(adst_t, [d16])
                    e = a + bb
                    e = jnp.where(e >= 0, e, 0.2 * e)
                    w16 = jnp.exp(e)
                    for i in range(16):
                        wspl = jnp.full((16,), w16[i])
                        di = d16[i]
                        erow = g * 16 + i
                        plsc.addupdate(
                            den_l.at[pl.ds(pl.multiple_of(di * 16, 16), 16)], wspl)
                        for j in range(RW // 16):
                            hv = hbuf[b, erow, pl.ds(j * 16, 16)]
                            plsc.addupdate(
                                out_l.at[pl.ds(pl.multiple_of(di * RW, 16) + j * 16, 16)],
                                hv * wspl)
                    return 0

                lax.fori_loop(0, PC // 16, group, 0)

                @pl.when(ch + 2 < nchunks)
                def _nxt():
                    issue_gather(k, ch + 2, b, sem)

            def outer(it, _):
                ch0 = it * 2

                @pl.when(ch0 < nchunks)
                def _b0():
                    process(ch0, 0, sem0)

                @pl.when(ch0 + 1 < nchunks)
                def _b1():
                    process(ch0 + 1, 1, sem1)

                return 0

            lax.fori_loop(0, (nchunks + 1) // 2, outer, 0)
            ob = pl.multiple_of((k * N_PAD + lo) * RW, 8)
            pltpu.sync_copy(out_l.at[pl.ds(0, NPW * RW)], acc_hbm.at[pl.ds(ob, NPW * RW)])
            db = pl.multiple_of((k * N_PAD + lo) * 16, 8)
            pltpu.sync_copy(den_l.at[pl.ds(0, NPW * 16)], den_hbm.at[pl.ds(db, NPW * 16)])
            return 0

        lax.fori_loop(0, heads, head_body, 0)

    return functools.partial(
        pl.kernel, body,
        out_type=[
            jax.ShapeDtypeStruct((heads * N_PAD * RW,), jnp.float32),
            jax.ShapeDtypeStruct((heads * N_PAD * 16,), jnp.float32),
        ],
        mesh=mesh,
        compiler_params=pltpu.CompilerParams(needs_layout_passes=False),
        scratch_types=[
            pltpu.VMEM((SC_CHUNK,), jnp.int32),
            pltpu.VMEM((SC_CHUNK,), jnp.int32),
            pltpu.VMEM((CAP,), jnp.int32),
            pltpu.VMEM((CAP,), jnp.int32),
            pltpu.VMEM((2, PC), jnp.int32),
            pltpu.VMEM((N_PAD,), jnp.float32),
            pltpu.VMEM((NPW + 16,), jnp.float32),
            pltpu.VMEM(((NPW + 1) * RW,), jnp.float32),
            pltpu.VMEM(((NPW + 1) * 16,), jnp.float32),
            pltpu.VMEM((2, PC, RW), jnp.float32),
            pltpu.SemaphoreType.DMA,
            pltpu.SemaphoreType.DMA,
        ],
    )()


def _edge_phase_sc(htab, asrc_t, adst_t, src, dst):
    heads = htab.shape[0] // N_PAD
    zeros = jnp.zeros(((NPW + 1) * RW,), jnp.float32)
    accf, denf = _sc_edge_kernel(heads)(src, dst, htab, asrc_t, adst_t, zeros)
    return accf.reshape(heads, N_PAD, RW), denf.reshape(heads, N_PAD, 16)


def _make_htab(h, heads, ch):
    hh = h.reshape(N, heads, ch).transpose(1, 0, 2)
    return jnp.pad(hh, ((0, 0), (0, N_PAD - N), (0, 0))).reshape(heads * N_PAD, RW)


def _pad_t(a, heads):
    return jnp.pad(a.T, ((0, 0), (0, N_PAD - N))).reshape(heads * N_PAD)


def kernel(x, edge_index, batch, W1, att_src1, att_dst1, b1, W2, att_src2, att_dst2, b2):
    src = edge_index[0].astype(jnp.int32)
    dst = edge_index[1].astype(jnp.int32)
    h1, asrc1, adst1 = _dense1(x, W1, att_src1, att_dst1)
    acc1, den1 = _edge_phase_sc(_make_htab(h1, HEADS, HID), _pad_t(asrc1, HEADS),
                                _pad_t(adst1, HEADS), src, dst)
    h2, asrc2, adst2 = _dense2(acc1, den1, b1, W2, att_src2, att_dst2)
    acc2, den2 = _edge_phase_sc(_make_htab(h2, 1, OUT), _pad_t(asrc2, 1),
                                _pad_t(adst2, 1), src, dst)
    return _pool(acc2, den2, b2, batch)


# speedup vs baseline: 5.6896x; 1.0426x over previous
import functools

import jax
import jax.numpy as jnp
from jax import lax
from jax.experimental import pallas as pl
from jax.experimental.pallas import tpu as pltpu

N = 10000
E = 320000
IN = 128
HID = 128
HEADS = 8
OUT = 128
G = 64

ROWS = 1000


def _dense1_body(x_ref, w_ref, asrc_w_ref, adst_w_ref, h_ref, asrc_ref, adst_ref):
    h = jnp.dot(x_ref[...], w_ref[...], preferred_element_type=jnp.float32)
    h_ref[...] = h
    hh = h.reshape(ROWS, HEADS, HID)
    asrc_ref[...] = jnp.sum(hh * asrc_w_ref[...][None], axis=-1)
    adst_ref[...] = jnp.sum(hh * adst_w_ref[...][None], axis=-1)


def _dense1(x, W1, att_src1, att_dst1):
    grid = N // ROWS
    return pl.pallas_call(
        _dense1_body,
        grid=(grid,),
        in_specs=[
            pl.BlockSpec((ROWS, IN), lambda i: (i, 0)),
            pl.BlockSpec((IN, HEADS * HID), lambda i: (0, 0)),
            pl.BlockSpec((HEADS, HID), lambda i: (0, 0)),
            pl.BlockSpec((HEADS, HID), lambda i: (0, 0)),
        ],
        out_specs=[
            pl.BlockSpec((ROWS, HEADS * HID), lambda i: (i, 0)),
            pl.BlockSpec((ROWS, HEADS), lambda i: (i, 0)),
            pl.BlockSpec((ROWS, HEADS), lambda i: (i, 0)),
        ],
        out_shape=[
            jax.ShapeDtypeStruct((N, HEADS * HID), jnp.float32),
            jax.ShapeDtypeStruct((N, HEADS), jnp.float32),
            jax.ShapeDtypeStruct((N, HEADS), jnp.float32),
        ],
    )(x, W1, att_src1, att_dst1)


def _dense2_body(acc_ref, den_ref, b1_ref, w2_ref, as2_ref, ad2_ref,
                 h2_ref, asrc2_ref, adst2_ref):
    a = acc_ref[...]
    hn = a / (den_ref[...][:, :, 0:1] + 1e-16)
    h1 = jnp.concatenate([hn[k] for k in range(HEADS)], axis=-1)
    h1 = h1 + b1_ref[...][None]
    h1 = jnp.where(h1 > 0, h1, jnp.exp(jnp.minimum(h1, 0.0)) - 1.0)
    h2 = jnp.dot(h1, w2_ref[...], preferred_element_type=jnp.float32)
    h2_ref[...] = h2
    asrc2_ref[...] = jnp.sum(h2 * as2_ref[...], axis=-1, keepdims=True)
    adst2_ref[...] = jnp.sum(h2 * ad2_ref[...], axis=-1, keepdims=True)


def _dense2(acc1, den1, b1, W2, att_src2, att_dst2):
    grid = N // ROWS
    return pl.pallas_call(
        _dense2_body,
        grid=(grid,),
        in_specs=[
            pl.BlockSpec((HEADS, ROWS, HID), lambda i: (0, i, 0)),
            pl.BlockSpec((HEADS, ROWS, 16), lambda i: (0, i, 0)),
            pl.BlockSpec((HEADS * HID,), lambda i: (0,)),
            pl.BlockSpec((HEADS * HID, OUT), lambda i: (0, 0)),
            pl.BlockSpec((1, OUT), lambda i: (0, 0)),
            pl.BlockSpec((1, OUT), lambda i: (0, 0)),
        ],
        out_specs=[
            pl.BlockSpec((ROWS, OUT), lambda i: (i, 0)),
            pl.BlockSpec((ROWS, 1), lambda i: (i, 0)),
            pl.BlockSpec((ROWS, 1), lambda i: (i, 0)),
        ],
        out_shape=[
            jax.ShapeDtypeStruct((N, OUT), jnp.float32),
            jax.ShapeDtypeStruct((N, 1), jnp.float32),
            jax.ShapeDtypeStruct((N, 1), jnp.float32),
        ],
    )(acc1, den1, b1, W2, att_src2, att_dst2)


def _pool_body(acc_ref, den_ref, b2_ref, batch_ref, out_ref):
    i = pl.program_id(0)
    a = acc_ref[...]
    h2 = a[0] / (den_ref[...][0, :, 0:1] + 1e-16) + b2_ref[...][None]
    seg = batch_ref[...].reshape(1, ROWS)
    gid = lax.broadcasted_iota(jnp.int32, (G, ROWS), 0)
    onehot = (seg == gid).astype(jnp.float32)
    sums = jnp.dot(onehot, h2, preferred_element_type=jnp.float32)
    counts = jnp.sum(onehot, axis=1, keepdims=True)

    @pl.when(i == 0)
    def _init():
        out_ref[...] = jnp.zeros_like(out_ref)

    out_ref[:, :OUT] += sums
    out_ref[:, OUT:] += counts


def _pool(acc2, den2, b2, batch):
    grid = N // ROWS
    out = pl.pallas_call(
        _pool_body,
        grid=(grid,),
        in_specs=[
            pl.BlockSpec((1, ROWS, OUT), lambda i: (0, i, 0)),
            pl.BlockSpec((1, ROWS, 16), lambda i: (0, i, 0)),
            pl.BlockSpec((OUT,), lambda i: (0,)),
            pl.BlockSpec((1, 1, ROWS), lambda i: (i, 0, 0)),
        ],
        out_specs=pl.BlockSpec((G, OUT + 1), lambda i: (0, 0)),
        out_shape=jax.ShapeDtypeStruct((G, OUT + 1), jnp.float32),
    )(acc2, den2, b2, batch.reshape(N // ROWS, 1, ROWS))
    sums = out[:, :OUT]
    counts = out[:, OUT:]
    return sums / jnp.clip(counts, 1.0)


NW = 32
NPW = 320
N_PAD = NW * NPW
RW = 128
CAPL = 960
CAP = 16 * CAPL
PC = 128
SC_CHUNK = 2000


def _sc_compact_kernel():
    from jax.experimental.pallas import tpu_sc as plsc

    mesh = plsc.VectorSubcoreMesh(core_axis_name="c", subcore_axis_name="s")

    def body(src_hbm, dst_hbm, srcc_hbm, dstc_hbm, meta_hbm,
             sbuf_src, sbuf_dst, src_c, dstl_c, meta_b):
        wid = lax.axis_index("s") * 2 + lax.axis_index("c")
        lo = wid * NPW

        zvi = jnp.zeros((16,), jnp.int32)
        sentinel = jnp.full((16,), NPW, jnp.int32)

        def zloop(i, _):
            src_c[pl.ds(pl.multiple_of(i * 16, 16), 16)] = zvi
            dstl_c[pl.ds(pl.multiple_of(i * 16, 16), 16)] = sentinel
            return 0

        lax.fori_loop(0, CAP // 16, zloop, 0)

        lane = lax.broadcasted_iota(jnp.int32, (16,), 0)
        ones = jnp.full((16,), 1, jnp.int32)
        zeros16 = jnp.zeros((16,), jnp.int32)

        def chunk_scan(ci, pos_v):
            base = pl.multiple_of(ci * SC_CHUNK, 8)
            pltpu.sync_copy(src_hbm.at[pl.ds(base, SC_CHUNK)], sbuf_src)
            pltpu.sync_copy(dst_hbm.at[pl.ds(base, SC_CHUNK)], sbuf_dst)

            def group(gi, pos_v):
                off = pl.multiple_of(gi * 16, 16)
                s16 = sbuf_src[pl.ds(off, 16)]
                d16 = sbuf_dst[pl.ds(off, 16)]
                m = (d16 >= lo) & (d16 < lo + NPW)
                addr = jnp.minimum(pos_v, CAPL - 1) * 16 + lane
                plsc.store_scatter(src_c, [addr], s16, mask=m)
                plsc.store_scatter(dstl_c, [addr], d16 - lo, mask=m)
                return pos_v + jnp.where(m, ones, zeros16)

            return lax.fori_loop(0, SC_CHUNK // 16, group, pos_v)

        pos_v = lax.fori_loop(0, E // SC_CHUNK, chunk_scan, zeros16)

        maxc = jnp.int32(0)
        for bit in (512, 256, 128, 64, 32, 16, 8, 4, 2, 1):
            t = maxc + bit
            c = plsc.all_reduce_population_count(pos_v >= t)
            maxc = jnp.where(c[0] > 0, t, maxc)
        maxc = jnp.minimum(maxc, CAPL)
        nchunks = (maxc + PC // 16 - 1) // (PC // 16)
        meta_b[pl.ds(0, 16)] = jnp.full((16,), nchunks, jnp.int32)

        cb = pl.multiple_of(wid * CAP, 8)
        pltpu.sync_copy(src_c, srcc_hbm.at[pl.ds(cb, CAP)])
        pltpu.sync_copy(dstl_c, dstc_hbm.at[pl.ds(cb, CAP)])
        pltpu.sync_copy(meta_b, meta_hbm.at[pl.ds(pl.multiple_of(wid * 16, 8), 16)])

    return functools.partial(
        pl.kernel, body,
        out_type=[
            jax.ShapeDtypeStruct((NW * CAP,), jnp.int32),
            jax.ShapeDtypeStruct((NW * CAP,), jnp.int32),
            jax.ShapeDtypeStruct((NW * 16,), jnp.int32),
        ],
        mesh=mesh,
        compiler_params=pltpu.CompilerParams(needs_layout_passes=False),
        scratch_types=[
            pltpu.VMEM((SC_CHUNK,), jnp.int32),
            pltpu.VMEM((SC_CHUNK,), jnp.int32),
            pltpu.VMEM((CAP,), jnp.int32),
            pltpu.VMEM((CAP,), jnp.int32),
            pltpu.VMEM((16,), jnp.int32),
        ],
    )()


def _sc_edge_kernel(heads):
    from jax.experimental.pallas import tpu_sc as plsc

    mesh = plsc.VectorSubcoreMesh(core_axis_name="c", subcore_axis_name="s")

    def body(srcc_hbm, dstc_hbm, meta_hbm, htab_hbm, asrc_hbm, adst_hbm,
             zeros_hbm, acc_hbm, den_hbm,
             src_c, dstl_c, meta_b, idxb, asrc_t, adst_t,
             out_l, den_l, hbuf, sem0, sem1):
        wid = lax.axis_index("s") * 2 + lax.axis_index("c")
        lo = wid * NPW

        cb = pl.multiple_of(wid * CAP, 8)
        pltpu.sync_copy(srcc_hbm.at[pl.ds(cb, CAP)], src_c)
        pltpu.sync_copy(dstc_hbm.at[pl.ds(cb, CAP)], dstl_c)
        pltpu.sync_copy(meta_hbm.at[pl.ds(pl.multiple_of(wid * 16, 8), 16)], meta_b)
        nchunks = meta_b[pl.ds(0, 16)][0]

        def issue_gather(k, ch, b, sem):
            cb = pl.multiple_of(ch * PC, 16)
            kbase = k * N_PAD
            for q in range(PC // 16):
                idxb[b, pl.ds(q * 16, 16)] = src_c[pl.ds(cb + q * 16, 16)] + kbase
            pltpu.make_async_copy(htab_hbm.at[idxb.at[b]], hbuf.at[b], sem).start()

        def wait_gather(k, ch, b, sem):
            pltpu.make_async_copy(htab_hbm.at[idxb.at[b]], hbuf.at[b], sem).wait()

        def head_body(k, _):
            kp = pl.multiple_of(k * N_PAD, 8)
            pltpu.sync_copy(asrc_hbm.at[pl.ds(kp, N_PAD)], asrc_t)
            pltpu.sync_copy(adst_hbm.at[pl.ds(kp + lo, NPW)], adst_t.at[pl.ds(0, NPW)])
            adst_t[pl.ds(NPW, 16)] = jnp.zeros((16,), jnp.float32)
            pltpu.sync_copy(zeros_hbm, out_l)
            pltpu.sync_copy(zeros_hbm.at[pl.ds(0, (NPW + 1) * 16)], den_l)

            @pl.when(nchunks > 0)
            def _p0():
                issue_gather(k, 0, 0, sem0)

            @pl.when(nchunks > 1)
            def _p1():
                issue_gather(k, 1, 1, sem1)

            def process(ch, b, sem):
                wait_gather(k, ch, b, sem)

                def group(g, _):
                    off = pl.multiple_of(ch * PC + g * 16, 16)
                    s16 = src_c[pl.ds(off, 16)]
                    d16 = dstl_c[pl.ds(off, 16)]
                    a = plsc.load_gather(asrc_t, [s16])
                    bb = plsc.load_gather(adst_t, [d16])
                    e = a + bb
                    e = jnp.where(e >= 0, e, 0.2 * e)
                    w16 = jnp.exp(e)
                    for i in range(16):
                        wspl = jnp.full((16,), w16[i])
                        di = d16[i]
                        erow = g * 16 + i
                        plsc.addupdate(
                            den_l.at[pl.ds(pl.multiple_of(di * 16, 16), 16)], wspl)
                        for j in range(RW // 16):
                            hv = hbuf[b, erow, pl.ds(j * 16, 16)]
                            plsc.addupdate(
                                out_l.at[pl.ds(pl.multiple_of(di * RW, 16) + j * 16, 16)],
                                hv * wspl)
                    return 0

                lax.fori_loop(0, PC // 16, group, 0)

                @pl.when(ch + 2 < nchunks)
                def _nxt():
                    issue_gather(k, ch + 2, b, sem)

            def outer(it, _):
                ch0 = it * 2

                @pl.when(ch0 < nchunks)
                def _b0():
                    process(ch0, 0, sem0)

                @pl.when(ch0 + 1 < nchunks)
                def _b1():
                    process(ch0 + 1, 1, sem1)

                return 0

            lax.fori_loop(0, (nchunks + 1) // 2, outer, 0)
            ob = pl.multiple_of((k * N_PAD + lo) * RW, 8)
            pltpu.sync_copy(out_l.at[pl.ds(0, NPW * RW)], acc_hbm.at[pl.ds(ob, NPW * RW)])
            db = pl.multiple_of((k * N_PAD + lo) * 16, 8)
            pltpu.sync_copy(den_l.at[pl.ds(0, NPW * 16)], den_hbm.at[pl.ds(db, NPW * 16)])
            return 0

        lax.fori_loop(0, heads, head_body, 0)

    return functools.partial(
        pl.kernel, body,
        out_type=[
            jax.ShapeDtypeStruct((heads * N_PAD * RW,), jnp.float32),
            jax.ShapeDtypeStruct((heads * N_PAD * 16,), jnp.float32),
        ],
        mesh=mesh,
        compiler_params=pltpu.CompilerParams(needs_layout_passes=False),
        scratch_types=[
            pltpu.VMEM((CAP,), jnp.int32),
            pltpu.VMEM((CAP,), jnp.int32),
            pltpu.VMEM((16,), jnp.int32),
            pltpu.VMEM((2, PC), jnp.int32),
            pltpu.VMEM((N_PAD,), jnp.float32),
            pltpu.VMEM((NPW + 16,), jnp.float32),
            pltpu.VMEM(((NPW + 1) * RW,), jnp.float32),
            pltpu.VMEM(((NPW + 1) * 16,), jnp.float32),
            pltpu.VMEM((2, PC, RW), jnp.float32),
            pltpu.SemaphoreType.DMA,
            pltpu.SemaphoreType.DMA,
        ],
    )()


def _edge_phase_sc(htab, asrc_t, adst_t, srcc, dstc, meta):
    heads = htab.shape[0] // N_PAD
    zeros = jnp.zeros(((NPW + 1) * RW,), jnp.float32)
    accf, denf = _sc_edge_kernel(heads)(srcc, dstc, meta, htab, asrc_t, adst_t, zeros)
    return accf.reshape(heads, N_PAD, RW), denf.reshape(heads, N_PAD, 16)


def _make_htab(h, heads, ch):
    hh = h.reshape(N, heads, ch).transpose(1, 0, 2)
    return jnp.pad(hh, ((0, 0), (0, N_PAD - N), (0, 0))).reshape(heads * N_PAD, RW)


def _pad_t(a, heads):
    return jnp.pad(a.T, ((0, 0), (0, N_PAD - N))).reshape(heads * N_PAD)


def kernel(x, edge_index, batch, W1, att_src1, att_dst1, b1, W2, att_src2, att_dst2, b2):
    src = edge_index[0].astype(jnp.int32)
    dst = edge_index[1].astype(jnp.int32)
    srcc, dstc, meta = _sc_compact_kernel()(src, dst)
    h1, asrc1, adst1 = _dense1(x, W1, att_src1, att_dst1)
    acc1, den1 = _edge_phase_sc(_make_htab(h1, HEADS, HID), _pad_t(asrc1, HEADS),
                                _pad_t(adst1, HEADS), srcc, dstc, meta)
    h2, asrc2, adst2 = _dense2(acc1, den1, b1, W2, att_src2, att_dst2)
    acc2, den2 = _edge_phase_sc(_make_htab(h2, 1, OUT), _pad_t(asrc2, 1),
                                _pad_t(adst2, 1), srcc, dstc, meta)
    return _pool(acc2, den2, b2, batch)
